# Initial kernel scaffold; baseline (speedup 1.0000x reference)
import jax
import jax.numpy as jnp
from jax.experimental import pallas as pl


def kernel(skills, times, labels, W, skill_base, alpha_skill_embeddings, ie_W1, ie_b1, ie_W2, ie_b2, ie_W3, ie_b3, ie_Wsk, ie_bsk, f_W1, f_b1, f_W2, f_b2, f_W3, f_b3, f_Wsk, f_bsk):
    raise NotImplementedError("write your pallas kernel here")


# speedup vs baseline: 13.4626x; 13.4626x over previous
import functools

import jax
import jax.numpy as jnp
from jax import lax
from jax.experimental import pallas as pl
from jax.experimental.pallas import tpu as pltpu
from jax.experimental.pallas import tpu_sc as plsc

_LAG = 5
_NC = 2
_NS = 16
_NW = _NC * _NS
_CH = 80


def _mk_job(wid):
    def job(table_hbm, idx_hbm, out_hbm, idx_v, per_w, bufs, sems):
        nbuf = len(bufs)
        base = wid * per_w
        nch = per_w // _CH
        pltpu.sync_copy(idx_hbm.at[pl.ds(base, per_w)], idx_v)

        def fire(c, buf, sem):
            pltpu.async_copy(
                table_hbm.at[idx_v.at[pl.ds(c * _CH, _CH)]], buf, sem)

        def drain(buf, sem):
            pltpu.make_async_copy(
                out_hbm.at[pl.ds(base, _CH)], buf, sem).wait()

        for b in range(nbuf):
            fire(b, bufs[b], sems[b])

        @pl.loop(0, nch, step=nbuf)
        def _(c):
            for b in range(nbuf):
                drain(bufs[b], sems[b])
                pltpu.sync_copy(
                    bufs[b], out_hbm.at[pl.ds(base + (c + b) * _CH, _CH)])

                @pl.when(c + b + nbuf < nch)
                def _(b=b):
                    fire(c + b + nbuf, bufs[b], sems[b])

    return job


def _sc_mesh():
    return plsc.VectorSubcoreMesh(
        core_axis_name="c", subcore_axis_name="s",
        num_cores=_NC, num_subcores=_NS)


def _sc_gather_emb(emb, basev, idxe, idxb):
    embd = emb.shape[1]
    epw = idxe.shape[0] // _NW
    bpw = idxb.shape[0] // _NW

    @functools.partial(
        pl.kernel,
        mesh=_sc_mesh(),
        compiler_params=pltpu.CompilerParams(use_tc_tiling_on_sc=False),
        out_type=[
            jax.ShapeDtypeStruct((idxe.shape[0], embd), jnp.float32),
            jax.ShapeDtypeStruct((idxb.shape[0],), jnp.float32),
        ],
        scratch_types=(
            [pltpu.VMEM((epw,), jnp.int32), pltpu.VMEM((bpw,), jnp.int32)]
            + [pltpu.VMEM((_CH, embd), jnp.float32)] * 4
            + [pltpu.VMEM((_CH,), jnp.float32)] * 4
            + [pltpu.SemaphoreType.DMA] * 4
        ),
    )
    def k(emb_hbm, base_hbm, idxe_hbm, idxb_hbm, oute_hbm, outb_hbm,
          idxe_v, idxb_v, r0, r1, r2, r3, s0, s1, s2, s3,
          m0, m1, m2, m3):
        wid = lax.axis_index("s") * _NC + lax.axis_index("c")
        job = _mk_job(wid)
        job(emb_hbm, idxe_hbm, oute_hbm, idxe_v, epw,
            [r0, r1, r2, r3], [m0, m1, m2, m3])
        job(base_hbm, idxb_hbm, outb_hbm, idxb_v, bpw,
            [s0, s1, s2, s3], [m0, m1, m2, m3])

    return k(emb, basev, idxe, idxb)


def _sc_gather_w(wflat, idxw):
    wpw = idxw.shape[0] // _NW

    @functools.partial(
        pl.kernel,
        mesh=_sc_mesh(),
        compiler_params=pltpu.CompilerParams(use_tc_tiling_on_sc=False),
        out_type=jax.ShapeDtypeStruct((idxw.shape[0],), jnp.float32),
        scratch_types=(
            [pltpu.VMEM((wpw,), jnp.int32)]
            + [pltpu.VMEM((_CH,), jnp.float32)] * 4
            + [pltpu.SemaphoreType.DMA] * 4
        ),
    )
    def k(wflat_hbm, idxw_hbm, outw_hbm, idxw_v,
          s0, s1, s2, s3, m0, m1, m2, m3):
        wid = lax.axis_index("s") * _NC + lax.axis_index("c")
        job = _mk_job(wid)
        job(wflat_hbm, idxw_hbm, outw_hbm, idxw_v, wpw,
            [s0, s1, s2, s3], [m0, m1, m2, m3])

    return k(wflat, idxw)


def _dot(a, b):
    return lax.dot_general(a, b, (((1,), (0,)), ((), ())),
                           preferred_element_type=jnp.float32)


def _dotb(a, b):
    return lax.dot_general(a.astype(jnp.bfloat16), b.astype(jnp.bfloat16),
                           (((1,), (0,)), ((), ())),
                           preferred_element_type=jnp.float32)


def _leaky(x):
    return jnp.maximum(x, 0.01 * x)


def _shift(x, s):
    return jnp.concatenate([jnp.zeros((s,) + x.shape[1:], x.dtype), x[:-s]],
                           axis=0)


def _tc_body(e_ref, aux1_ref, aux2_ref, aux3_ref, wm_ref, rv_ref, fb_ref,
             out_ref):
    a1 = aux1_ref[...]
    a2 = aux2_ref[...]
    a3 = aux3_ref[...]
    wm = wm_ref[...]
    w1a, w1b = wm[0:64], wm[64:128]
    wska, wskb = wm[128:192], wm[192:256]
    w2, w3 = wm[256:320], wm[320:384]
    fw1, fw2 = wm[384:448], wm[448:512]
    rv = rv_ref[...]
    b1, b2, b3e, bsk = rv[0:1], rv[1:2], rv[2:3], rv[3:4]
    fb1, fb2, w3row, wskrow = rv[4:5], rv[5:6], rv[6:7], rv[7:8]

    inv_log5 = 1.0 / jnp.log(5.0)
    t5 = a1[:, 0:5]
    tt = a1[:, 5:6]
    dt5 = jnp.abs(tt - t5)
    e5 = jnp.exp(-(jnp.log(dt5 + 1e-10) * inv_log5))
    wgt5 = jnp.where(a3[:, 0:5] != 0.0, a2[:, 0:5], 0.0) * e5
    lab = a1[:, 6:7]
    lab = jnp.where(lab == 0.0, -1.0, jnp.where(lab == -1.0, 0.0, lab))
    bc = jnp.concatenate([wgt5, lab, a1[:, 7:8], jnp.zeros_like(lab)],
                         axis=1)

    iota8 = lax.broadcasted_iota(jnp.int32, (8, 64), 0)

    def _sel(c):
        return (iota8 == c).astype(jnp.float32)

    lab64 = _dot(bc, _sel(5))
    base64 = _dot(bc, _sel(6))
    wsum64 = _dot(bc, (iota8 < _LAG).astype(jnp.float32))

    e = e_ref[...]
    el = e * lab64
    p = (_dotb(e, w1a) + b1).astype(jnp.bfloat16)
    q = _dotb(el, w1b).astype(jnp.bfloat16)
    stc = _dotb(e, wska) + b3e + bsk
    sh = _dotb(el, wskb).astype(jnp.bfloat16)

    h2acc = jnp.zeros_like(p)
    shacc = jnp.zeros_like(p)
    for l in range(_LAG):
        s = _LAG - l
        wl = _dot(bc, _sel(l))
        h1 = _leaky(p + _shift(q, s))
        h2 = _leaky(_dotb(h1, w2) + b2)
        h2acc = h2acc + wl * h2
        shacc = shacc + wl * _shift(sh, s).astype(jnp.float32)
    ce = _dotb(h2acc, w3) + wsum64 * stc + shacc + base64

    g1 = _leaky(_dotb(ce, fw1) + fb1)
    g2 = _leaky(_dotb(g1, fw2) + fb2)
    o = (jnp.sum(g2 * w3row, axis=1, keepdims=True)
         + jnp.sum(ce * wskrow, axis=1, keepdims=True) + fb_ref[...])
    out_ref[...] = jax.nn.sigmoid(o)


def kernel(skills, times, labels, W, skill_base, alpha_skill_embeddings,
           ie_W1, ie_b1, ie_W2, ie_b2, ie_W3, ie_b3, ie_Wsk, ie_bsk,
           f_W1, f_b1, f_W2, f_b2, f_W3, f_b3, f_Wsk, f_bsk):
    bs, ts = skills.shape
    n = W.shape[0]
    rf = bs * ts

    emb_table = alpha_skill_embeddings[0]
    basev = skill_base[0]
    wflat = W.reshape(-1)

    idxe = skills.reshape(-1)
    idxw_groups = []
    for l in range(_LAG):
        s = _LAG - l
        hist = jnp.pad(skills, ((0, 0), (s, 0)))[:, :ts]
        idxw_groups.append((hist * n + skills).reshape(-1))
    idxw = jnp.concatenate(idxw_groups)

    outw = _sc_gather_w(wflat, idxw)
    oute, outb = _sc_gather_emb(emb_table, basev, idxe, idxe)

    zcol = jnp.zeros((rf,), jnp.float32)
    tcols, dcols = [], []
    for l in range(_LAG):
        s = _LAG - l
        tcols.append(jnp.pad(times, ((0, 0), (s, 0)))[:, :ts].reshape(-1))
        hist = jnp.pad(skills, ((0, 0), (s, 0)))[:, :ts]
        dcols.append((hist - skills).astype(jnp.float32).reshape(-1))
    aux1 = jnp.stack(
        tcols + [times.reshape(-1), labels.astype(jnp.float32).reshape(-1),
                 outb], axis=1)
    aux2 = jnp.stack(
        [outw[l * rf:(l + 1) * rf] for l in range(_LAG)] + [zcol] * 3,
        axis=1)
    aux3 = jnp.stack(dcols + [zcol] * 3, axis=1)

    wmat = jnp.concatenate([
        ie_W1[:64], ie_W1[64:], ie_Wsk[:64], ie_Wsk[64:],
        ie_W2, ie_W3, f_W1, f_W2], axis=0)
    rvec = jnp.stack([
        ie_b1, ie_b2, ie_b3, ie_bsk, f_b1, f_b2,
        f_W3[:, 0], f_Wsk[:, 0]], axis=0)
    fbias = (f_b3 + f_bsk).reshape(1, 1)

    bt = 3200
    nt = rf // bt
    embd = emb_table.shape[1]

    out = pl.pallas_call(
        _tc_body,
        grid=(nt,),
        in_specs=[
            pl.BlockSpec((bt, embd), lambda i: (i, 0)),
            pl.BlockSpec((bt, 8), lambda i: (i, 0)),
            pl.BlockSpec((bt, 8), lambda i: (i, 0)),
            pl.BlockSpec((bt, 8), lambda i: (i, 0)),
            pl.BlockSpec((512, 64), lambda i: (0, 0)),
            pl.BlockSpec((8, 64), lambda i: (0, 0)),
            pl.BlockSpec((1, 1), lambda i: (0, 0)),
        ],
        out_specs=pl.BlockSpec((bt, 1), lambda i: (i, 0)),
        out_shape=jax.ShapeDtypeStruct((rf, 1), jnp.float32),
    )(oute, aux1, aux2, aux3, wmat, rvec, fbias)

    return out.reshape(bs, ts)[:, _LAG:]
